# baseline (speedup 1.0000x reference)
import functools

import jax
import jax.numpy as jnp
from jax import lax
from jax.experimental import pallas as pl
from jax.experimental.pallas import tpu as pltpu
from jax.experimental.pallas import tpu_sc as plsc

VOCAB_SIZE = 1000000
EMBED_DIM = 32
BATCH = 4096
POS = 200

NUM_WORKERS = 32

TCOL = 128
NFULL = VOCAB_SIZE // TCOL
TAIL = VOCAB_SIZE - NFULL * TCOL
TMAIN = (NFULL // NUM_WORKERS) & ~1
NPAIRS = TMAIN // 2

HALF = POS // 2
B_PER_W = BATCH // NUM_WORKERS
NB = 8
GROUPS = B_PER_W // NB

_mesh = plsc.VectorSubcoreMesh(core_axis_name="c", subcore_axis_name="s")
_IOTA = None


@functools.partial(
    pl.kernel,
    mesh=_mesh,
    out_type=jax.ShapeDtypeStruct((VOCAB_SIZE * EMBED_DIM,), jnp.float32),
    scratch_types=[
        pltpu.VMEM((EMBED_DIM, TCOL + 1), jnp.float32),
        pltpu.VMEM((EMBED_DIM, TCOL + 1), jnp.float32),
        pltpu.VMEM((TCOL * EMBED_DIM,), jnp.float32),
        pltpu.VMEM((TCOL * EMBED_DIM,), jnp.float32),
        pltpu.SemaphoreType.DMA,
        pltpu.SemaphoreType.DMA,
        pltpu.SemaphoreType.DMA,
        pltpu.SemaphoreType.DMA,
    ],
    compiler_params=pltpu.CompilerParams(
        use_tc_tiling_on_sc=True, needs_layout_passes=False),
)
def _transpose_table(wt_hbm, wt2_hbm, out_hbm, in0, in1, ob0, ob1,
                     si0, si1, so0, so1):
    wid = lax.axis_index("s") * 2 + lax.axis_index("c")
    iota = lax.iota(jnp.int32, 16)

    def fire_in(t, buf, sem):
        j = wid + NUM_WORKERS * t
        pltpu.async_copy(wt_hbm.at[:, pl.ds(j * TCOL, TCOL)],
                         buf.at[:, pl.ds(0, TCOL)], sem)

    def drain(buf, sem):
        pltpu.make_async_copy(wt_hbm.at[:, pl.ds(0, TCOL)],
                              buf.at[:, pl.ds(0, TCOL)], sem).wait()

    def drain_out(buf, sem):
        pltpu.make_async_copy(out_hbm.at[pl.ds(0, TCOL * EMBED_DIM)], buf,
                              sem).wait()

    def transpose(inbuf, outbuf, ncols):
        rows = [iota, iota + 16]
        for l in range(ncols):
            lvec = jnp.full((16,), l, jnp.int32)
            for h in range(2):
                vals = plsc.load_gather(inbuf, [rows[h], lvec])
                outbuf[pl.ds(l * EMBED_DIM + 16 * h, 16)] = vals

    def fire_out(t, buf, sem):
        j = wid + NUM_WORKERS * t
        pltpu.async_copy(
            buf, out_hbm.at[pl.ds(j * (TCOL * EMBED_DIM), TCOL * EMBED_DIM)],
            sem)

    fire_in(0, in0, si0)
    fire_in(1, in1, si1)

    def body(k, carry):
        t0 = 2 * k
        drain(in0, si0)
        transpose(in0, ob0, TCOL)

        @pl.when(k > 0)
        def _():
            drain_out(ob0, so0)

        fire_out(t0, ob0, so0)

        @pl.when(k < NPAIRS - 1)
        def _():
            fire_in(t0 + 2, in0, si0)

        drain(in1, si1)
        transpose(in1, ob1, TCOL)

        @pl.when(k > 0)
        def _():
            drain_out(ob1, so1)

        fire_out(t0 + 1, ob1, so1)

        @pl.when(k < NPAIRS - 1)
        def _():
            fire_in(t0 + 3, in1, si1)

        return carry

    lax.fori_loop(0, NPAIRS, body, 0)
    drain_out(ob0, so0)
    drain_out(ob1, so1)

    nrem = NFULL - TMAIN * NUM_WORKERS

    @pl.when(wid < nrem)
    def _():
        j = TMAIN * NUM_WORKERS + wid
        pltpu.sync_copy(wt_hbm.at[:, pl.ds(j * TCOL, TCOL)],
                        in0.at[:, pl.ds(0, TCOL)])
        transpose(in0, ob0, TCOL)
        pltpu.sync_copy(
            ob0, out_hbm.at[pl.ds(j * (TCOL * EMBED_DIM), TCOL * EMBED_DIM)])

    @pl.when(wid == nrem)
    def _():
        pltpu.sync_copy(wt2_hbm, in0.at[:, pl.ds(0, TCOL)])
        transpose(in0, ob0, TCOL)
        pltpu.sync_copy(
            ob0,
            out_hbm.at[pl.ds((VOCAB_SIZE - TCOL) * EMBED_DIM,
                             TCOL * EMBED_DIM)])


@functools.partial(
    pl.kernel,
    mesh=_mesh,
    out_type=jax.ShapeDtypeStruct((BATCH, POS, EMBED_DIM), jnp.float32),
    scratch_types=[
        pltpu.VMEM((2 * B_PER_W, HALF), jnp.int32),
        pltpu.VMEM((NB, POS, EMBED_DIM), jnp.float32),
        pltpu.SemaphoreType.DMA,
    ],
    compiler_params=pltpu.CompilerParams(use_tc_tiling_on_sc=False),
)
def _embed_gather(idx_hbm, table_hbm, out_hbm, idx_v, buf, sem):
    wid = lax.axis_index("s") * 2 + lax.axis_index("c")
    bbase = wid * B_PER_W
    pltpu.sync_copy(idx_hbm.at[pl.ds(2 * bbase, 2 * B_PER_W)], idx_v)

    def body(g, carry):
        for ib in range(NB):
            for h in range(2):
                pltpu.async_copy(
                    table_hbm.at[idx_v.at[2 * (g * NB + ib) + h]],
                    buf.at[ib, pl.ds(h * HALF, HALF)],
                    sem,
                )
        pltpu.make_async_copy(out_hbm.at[pl.ds(0, NB)], buf, sem).wait()
        pltpu.sync_copy(buf, out_hbm.at[pl.ds(bbase + g * NB, NB)])
        return carry

    lax.fori_loop(0, GROUPS, body, 0)


def _tc_transpose_body(in_ref, out_ref):
    t = in_ref[...].T.reshape(TCOL, 4, EMBED_DIM)
    out_ref[...] = jnp.concatenate([t[:, q, :] for q in range(4)], axis=1)


_TCW = 512
_tc_transpose = pl.pallas_call(
    _tc_transpose_body,
    grid=((VOCAB_SIZE + _TCW - 1) // _TCW,),
    in_specs=[pl.BlockSpec((EMBED_DIM, _TCW), lambda j: (0, j))],
    out_specs=pl.BlockSpec((TCOL, TCOL), lambda j: (j, 0)),
    out_shape=jax.ShapeDtypeStruct(
        (VOCAB_SIZE * EMBED_DIM // TCOL, TCOL), jnp.float32),
)


def kernel(x, weight):
    idx = x.reshape(2 * BATCH, HALF).astype(jnp.int32)
    table128 = _tc_transpose(weight.T)
    table = table128.reshape(VOCAB_SIZE, EMBED_DIM)
    return _embed_gather(idx, table)


# speedup vs baseline: 1.6755x; 1.6755x over previous
import functools

import jax
import jax.numpy as jnp
from jax import lax
from jax.experimental import pallas as pl
from jax.experimental.pallas import tpu as pltpu
from jax.experimental.pallas import tpu_sc as plsc

VOCAB_SIZE = 1000000
EMBED_DIM = 32
BATCH = 4096
POS = 200

NUM_WORKERS = 32

TCOL = 128
NFULL = VOCAB_SIZE // TCOL
TAIL = VOCAB_SIZE - NFULL * TCOL
TMAIN = (NFULL // NUM_WORKERS) & ~1
NPAIRS = TMAIN // 2

HALF = POS // 2
B_PER_W = BATCH // NUM_WORKERS
NB = 8
GROUPS = B_PER_W // NB

_mesh = plsc.VectorSubcoreMesh(core_axis_name="c", subcore_axis_name="s")
_IOTA = None


@functools.partial(
    pl.kernel,
    mesh=_mesh,
    out_type=jax.ShapeDtypeStruct((VOCAB_SIZE * EMBED_DIM,), jnp.float32),
    scratch_types=[
        pltpu.VMEM((EMBED_DIM, TCOL + 1), jnp.float32),
        pltpu.VMEM((EMBED_DIM, TCOL + 1), jnp.float32),
        pltpu.VMEM((TCOL * EMBED_DIM,), jnp.float32),
        pltpu.VMEM((TCOL * EMBED_DIM,), jnp.float32),
        pltpu.SemaphoreType.DMA,
        pltpu.SemaphoreType.DMA,
        pltpu.SemaphoreType.DMA,
        pltpu.SemaphoreType.DMA,
    ],
    compiler_params=pltpu.CompilerParams(
        use_tc_tiling_on_sc=True, needs_layout_passes=False),
)
def _transpose_table(wt_hbm, wt2_hbm, out_hbm, in0, in1, ob0, ob1,
                     si0, si1, so0, so1):
    wid = lax.axis_index("s") * 2 + lax.axis_index("c")
    iota = lax.iota(jnp.int32, 16)

    def fire_in(t, buf, sem):
        j = wid + NUM_WORKERS * t
        pltpu.async_copy(wt_hbm.at[:, pl.ds(j * TCOL, TCOL)],
                         buf.at[:, pl.ds(0, TCOL)], sem)

    def drain(buf, sem):
        pltpu.make_async_copy(wt_hbm.at[:, pl.ds(0, TCOL)],
                              buf.at[:, pl.ds(0, TCOL)], sem).wait()

    def drain_out(buf, sem):
        pltpu.make_async_copy(out_hbm.at[pl.ds(0, TCOL * EMBED_DIM)], buf,
                              sem).wait()

    def transpose(inbuf, outbuf, ncols):
        rows = [iota, iota + 16]
        for l in range(ncols):
            lvec = jnp.full((16,), l, jnp.int32)
            for h in range(2):
                vals = plsc.load_gather(inbuf, [rows[h], lvec])
                outbuf[pl.ds(l * EMBED_DIM + 16 * h, 16)] = vals

    def fire_out(t, buf, sem):
        j = wid + NUM_WORKERS * t
        pltpu.async_copy(
            buf, out_hbm.at[pl.ds(j * (TCOL * EMBED_DIM), TCOL * EMBED_DIM)],
            sem)

    fire_in(0, in0, si0)
    fire_in(1, in1, si1)

    def body(k, carry):
        t0 = 2 * k
        drain(in0, si0)
        transpose(in0, ob0, TCOL)

        @pl.when(k > 0)
        def _():
            drain_out(ob0, so0)

        fire_out(t0, ob0, so0)

        @pl.when(k < NPAIRS - 1)
        def _():
            fire_in(t0 + 2, in0, si0)

        drain(in1, si1)
        transpose(in1, ob1, TCOL)

        @pl.when(k > 0)
        def _():
            drain_out(ob1, so1)

        fire_out(t0 + 1, ob1, so1)

        @pl.when(k < NPAIRS - 1)
        def _():
            fire_in(t0 + 3, in1, si1)

        return carry

    lax.fori_loop(0, NPAIRS, body, 0)
    drain_out(ob0, so0)
    drain_out(ob1, so1)

    nrem = NFULL - TMAIN * NUM_WORKERS

    @pl.when(wid < nrem)
    def _():
        j = TMAIN * NUM_WORKERS + wid
        pltpu.sync_copy(wt_hbm.at[:, pl.ds(j * TCOL, TCOL)],
                        in0.at[:, pl.ds(0, TCOL)])
        transpose(in0, ob0, TCOL)
        pltpu.sync_copy(
            ob0, out_hbm.at[pl.ds(j * (TCOL * EMBED_DIM), TCOL * EMBED_DIM)])

    @pl.when(wid == nrem)
    def _():
        pltpu.sync_copy(wt2_hbm, in0.at[:, pl.ds(0, TCOL)])
        transpose(in0, ob0, TCOL)
        pltpu.sync_copy(
            ob0,
            out_hbm.at[pl.ds((VOCAB_SIZE - TCOL) * EMBED_DIM,
                             TCOL * EMBED_DIM)])


@functools.partial(
    pl.kernel,
    mesh=_mesh,
    out_type=jax.ShapeDtypeStruct((BATCH, POS, EMBED_DIM), jnp.float32),
    scratch_types=[
        pltpu.VMEM((2 * B_PER_W, HALF), jnp.int32),
        pltpu.VMEM((NB, POS, EMBED_DIM), jnp.float32),
        pltpu.SemaphoreType.DMA,
    ],
    compiler_params=pltpu.CompilerParams(use_tc_tiling_on_sc=False),
)
def _embed_gather(idx_hbm, table_hbm, out_hbm, idx_v, buf, sem):
    wid = lax.axis_index("s") * 2 + lax.axis_index("c")
    bbase = wid * B_PER_W
    pltpu.sync_copy(idx_hbm.at[pl.ds(2 * bbase, 2 * B_PER_W)], idx_v)

    def body(g, carry):
        for ib in range(NB):
            for h in range(2):
                pltpu.async_copy(
                    table_hbm.at[idx_v.at[2 * (g * NB + ib) + h]],
                    buf.at[ib, pl.ds(h * HALF, HALF)],
                    sem,
                )
        pltpu.make_async_copy(out_hbm.at[pl.ds(0, NB)], buf, sem).wait()
        pltpu.sync_copy(buf, out_hbm.at[pl.ds(bbase + g * NB, NB)])
        return carry

    lax.fori_loop(0, GROUPS, body, 0)


_TCW = 2048


def _tc_transpose_body(in_ref, out_ref):
    x = in_ref[...]
    r = lax.broadcasted_iota(jnp.int32, (EMBED_DIM, EMBED_DIM), 0)
    c = lax.broadcasted_iota(jnp.int32, (EMBED_DIM, EMBED_DIM), 1)
    ident = jnp.where(r == c, 1.0, 0.0).astype(jnp.float32)
    t = lax.dot_general(x, ident, (((0,), (0,)), ((), ())),
                        preferred_element_type=jnp.float32)
    t = t.reshape(_TCW // 4, 4, EMBED_DIM)
    out_ref[...] = jnp.concatenate([t[:, q, :] for q in range(4)], axis=1)


_tc_transpose = pl.pallas_call(
    _tc_transpose_body,
    grid=((VOCAB_SIZE + _TCW - 1) // _TCW,),
    in_specs=[pl.BlockSpec((EMBED_DIM, _TCW), lambda j: (0, j))],
    out_specs=pl.BlockSpec((_TCW // 4, TCOL), lambda j: (j, 0)),
    out_shape=jax.ShapeDtypeStruct(
        (VOCAB_SIZE * EMBED_DIM // TCOL, TCOL), jnp.float32),
)


def kernel(x, weight):
    idx = x.reshape(2 * BATCH, HALF).astype(jnp.int32)
    table128 = _tc_transpose(weight.T)
    table = table128.reshape(VOCAB_SIZE, EMBED_DIM)
    return _embed_gather(idx, table)


# speedup vs baseline: 1.7144x; 1.0232x over previous
import functools

import jax
import jax.numpy as jnp
from jax import lax
from jax.experimental import pallas as pl
from jax.experimental.pallas import tpu as pltpu
from jax.experimental.pallas import tpu_sc as plsc

VOCAB_SIZE = 1000000
EMBED_DIM = 32
BATCH = 4096
POS = 200

HALF = POS // 2
NUM_WORKERS = 32
B_PER_W = BATCH // NUM_WORKERS
NB = 8
GROUPS = B_PER_W // NB

_mesh = plsc.VectorSubcoreMesh(core_axis_name="c", subcore_axis_name="s")


@functools.partial(
    pl.kernel,
    mesh=_mesh,
    out_type=jax.ShapeDtypeStruct((BATCH, POS, EMBED_DIM), jnp.float32),
    scratch_types=[
        pltpu.VMEM((2 * B_PER_W, HALF), jnp.int32),
        pltpu.VMEM((NB, POS, EMBED_DIM), jnp.float32),
        pltpu.SemaphoreType.DMA,
    ],
    compiler_params=pltpu.CompilerParams(use_tc_tiling_on_sc=False),
)
def _embed_gather(idx_hbm, table_hbm, out_hbm, idx_v, buf, sem):
    wid = lax.axis_index("s") * 2 + lax.axis_index("c")
    bbase = wid * B_PER_W
    pltpu.sync_copy(idx_hbm.at[pl.ds(2 * bbase, 2 * B_PER_W)], idx_v)

    def body(g, carry):
        for ib in range(NB):
            for h in range(2):
                pltpu.async_copy(
                    table_hbm.at[idx_v.at[2 * (g * NB + ib) + h]],
                    buf.at[ib, pl.ds(h * HALF, HALF)],
                    sem,
                )
        pltpu.make_async_copy(out_hbm.at[pl.ds(0, NB)], buf, sem).wait()
        pltpu.sync_copy(buf, out_hbm.at[pl.ds(bbase + g * NB, NB)])
        return carry

    lax.fori_loop(0, GROUPS, body, 0)


def kernel(x, weight):
    idx = x.reshape(2 * BATCH, HALF).astype(jnp.int32)
    return _embed_gather(idx, weight)
